# baseline (speedup 1.0000x reference)
import functools
import math

import jax
import jax.numpy as jnp
import numpy as np
from jax import lax
from jax.experimental import pallas as pl
from jax.experimental.pallas import tpu as pltpu
from jax.experimental.pallas import tpu_sc as plsc

_WINDOW_SIZE = 10


def _prep_body(ts_ref, emb_ref, st_ref, table_ref, *, num_bins):
    ts = ts_ref[...]
    rt = (ts - ts[:, 0:1]).astype(jnp.float32)
    mx = jnp.max(rt)
    st_ref[...] = jnp.clip(rt / mx, 0.0, float(num_bins - 1)).astype(jnp.int32)
    j = lax.broadcasted_iota(jnp.int32, (num_bins, num_bins), 1).astype(jnp.float32)
    k = lax.broadcasted_iota(jnp.int32, (num_bins, num_bins), 0).astype(jnp.float32)
    w = jnp.cos((math.pi / _WINDOW_SIZE) * (j - k)) ** 2
    wsum = jnp.sum(w, axis=1, keepdims=True)
    table_ref[...] = emb_ref[...] * wsum


def _make_sc_lookup(n_rows, d, n_chunks, chunk):
    info = plsc.get_sparse_core_info()
    nc, ns = info.num_cores, info.num_subcores
    nw = nc * ns
    rows_per_w = n_rows // nw
    assert rows_per_w == n_chunks * chunk
    mesh = plsc.VectorSubcoreMesh(core_axis_name="c", subcore_axis_name="s")

    @functools.partial(
        pl.kernel,
        mesh=mesh,
        out_type=jax.ShapeDtypeStruct((n_rows, d), jnp.float32),
        scratch_types=[
            pltpu.VMEM((n_chunks, chunk), jnp.int32),
            pltpu.VMEM((rows_per_w, d), jnp.float32),
            pltpu.SemaphoreType.DMA,
        ],
        compiler_params=pltpu.CompilerParams(use_tc_tiling_on_sc=False),
    )
    def sc_lookup(table_hbm, idx_hbm, out_hbm, idx_v, rows_v, sem):
        wid = lax.axis_index("s") * nc + lax.axis_index("c")
        pltpu.sync_copy(idx_hbm.at[pl.ds(wid * n_chunks, n_chunks)], idx_v)
        copies = []
        for j in range(n_chunks):
            copies.append(
                pltpu.async_copy(
                    table_hbm.at[idx_v.at[j]],
                    rows_v.at[pl.ds(j * chunk, chunk)],
                    sem,
                )
            )
        for cp in copies:
            cp.wait()
        pltpu.sync_copy(rows_v, out_hbm.at[pl.ds(wid * rows_per_w, rows_per_w)])

    return sc_lookup


def _make_sc_lookup_1stream(n_rows, d):
    info = plsc.get_sparse_core_info()
    nc, ns = info.num_cores, info.num_subcores
    nw = nc * ns
    rows_per_w = n_rows // nw
    mesh = plsc.VectorSubcoreMesh(core_axis_name="c", subcore_axis_name="s")

    @functools.partial(
        pl.kernel,
        mesh=mesh,
        out_type=jax.ShapeDtypeStruct((n_rows, d), jnp.float32),
        scratch_types=[
            pltpu.VMEM((rows_per_w,), jnp.int32),
            pltpu.VMEM((rows_per_w, d), jnp.float32),
            pltpu.SemaphoreType.DMA,
        ],
        compiler_params=pltpu.CompilerParams(use_tc_tiling_on_sc=False),
    )
    def sc_lookup(table_hbm, idx_hbm, out_hbm, idx_v, rows_v, sem):
        wid = lax.axis_index("s") * nc + lax.axis_index("c")
        base = wid * rows_per_w
        pltpu.sync_copy(idx_hbm.at[pl.ds(base, rows_per_w)], idx_v)
        pltpu.async_copy(table_hbm.at[idx_v], rows_v, sem).wait()
        pltpu.sync_copy(rows_v, out_hbm.at[pl.ds(base, rows_per_w)])

    return sc_lookup


def kernel(timestamps, time_embedding):
    b, s = timestamps.shape
    num_bins, d = time_embedding.shape
    n_rows = b * s
    prep = pl.pallas_call(
        functools.partial(_prep_body, num_bins=num_bins),
        out_shape=(
            jax.ShapeDtypeStruct((b, s), jnp.int32),
            jax.ShapeDtypeStruct((num_bins, d), jnp.float32),
        ),
    )
    st, table = prep(timestamps, time_embedding)
    idx = st.reshape(n_rows)
    out = _make_sc_lookup_1stream(n_rows, d)(table, idx)
    return out.reshape(b, s, d)


# speedup vs baseline: 6.9030x; 6.9030x over previous
import functools
import math

import jax
import jax.numpy as jnp
import numpy as np
from jax import lax
from jax.experimental import pallas as pl
from jax.experimental.pallas import tpu as pltpu
from jax.experimental.pallas import tpu_sc as plsc

_WINDOW_SIZE = 10


def _prep_body(ts_ref, emb_ref, t_ref, tt_ref, *, num_bins, s, d):
    ts = ts_ref[...]
    rt = (ts - ts[:, 0:1]).astype(jnp.float32)
    mx = jnp.max(rt)
    st = jnp.clip(rt / mx, 0.0, float(num_bins - 1)).astype(jnp.int32)
    t_ref[...] = jnp.sum((st == 0).astype(jnp.int32), axis=1, keepdims=True)
    j = lax.broadcasted_iota(jnp.int32, (num_bins, num_bins), 1).astype(jnp.float32)
    k = lax.broadcasted_iota(jnp.int32, (num_bins, num_bins), 0).astype(jnp.float32)
    w = jnp.cos((math.pi / _WINDOW_SIZE) * (j - k)) ** 2
    wsum = jnp.sum(w, axis=1, keepdims=True)
    l = s * d
    li2 = lax.broadcasted_iota(jnp.int32, (d, l), 1)
    di = lax.broadcasted_iota(jnp.int32, (d, l), 0)
    c_mat = (li2 % d == di).astype(jnp.float32)
    r0l = jnp.dot(emb_ref[0:1, :] * wsum[0, 0], c_mat,
                  preferred_element_type=jnp.float32)
    r1l = jnp.dot(emb_ref[1:2, :] * wsum[1, 0], c_mat,
                  preferred_element_type=jnp.float32)
    ti = lax.broadcasted_iota(jnp.int32, (s + 1, l), 0)
    si = lax.broadcasted_iota(jnp.int32, (s + 1, l), 1) // d
    tt_ref[...] = jnp.where(si < ti, r0l, r1l)


def _make_sc_lookup(n_b, row_w):
    info = plsc.get_sparse_core_info()
    nc, ns = info.num_cores, info.num_subcores
    nw = nc * ns
    rows_per_w = n_b // nw
    mesh = plsc.VectorSubcoreMesh(core_axis_name="c", subcore_axis_name="s")

    @functools.partial(
        pl.kernel,
        mesh=mesh,
        out_type=jax.ShapeDtypeStruct((n_b, row_w), jnp.float32),
        scratch_types=[
            pltpu.VMEM((rows_per_w,), jnp.int32),
            pltpu.VMEM((rows_per_w, row_w), jnp.float32),
            pltpu.SemaphoreType.DMA,
        ],
        compiler_params=pltpu.CompilerParams(use_tc_tiling_on_sc=False),
    )
    def sc_lookup(tt_hbm, idx_hbm, out_hbm, idx_v, rows_v, sem):
        wid = lax.axis_index("s") * nc + lax.axis_index("c")
        base = wid * rows_per_w
        pltpu.sync_copy(idx_hbm.at[pl.ds(base, rows_per_w)], idx_v)
        pltpu.async_copy(tt_hbm.at[idx_v], rows_v, sem).wait()
        pltpu.sync_copy(rows_v, out_hbm.at[pl.ds(base, rows_per_w)])

    return sc_lookup


def kernel(timestamps, time_embedding):
    b, s = timestamps.shape
    num_bins, d = time_embedding.shape
    prep = pl.pallas_call(
        functools.partial(_prep_body, num_bins=num_bins, s=s, d=d),
        out_shape=(
            jax.ShapeDtypeStruct((b, 1), jnp.int32),
            jax.ShapeDtypeStruct((s + 1, s * d), jnp.float32),
        ),
    )
    t, tt = prep(timestamps, time_embedding)
    out = _make_sc_lookup(b, s * d)(tt, t.reshape(b))
    return out.reshape(b, s, d)
